# baseline (speedup 1.0000x reference)
import jax
import jax.numpy as jnp
from jax.experimental import pallas as pl
from jax.experimental.pallas import tpu as pltpu

EPS_DIAG = 1e-06
T_TILE = 2048


def _cln_kernel(xr_ref, xi_ref, wb_ref, yr_ref, yi_ref):
    t = pl.program_id(1)
    xr = xr_ref[0, :, pl.ds(t * T_TILE, T_TILE)]
    xi = xi_ref[0, :, pl.ds(t * T_TILE, T_TILE)]
    c_dim = xr.shape[0]
    inv_c = 1.0 / c_dim

    mu_r = jnp.sum(xr, axis=0, keepdims=True) * inv_c
    mu_i = jnp.sum(xi, axis=0, keepdims=True) * inv_c
    xc_r = xr - mu_r
    xc_i = xi - mu_i

    a = jnp.sum(xc_r * xc_r, axis=0, keepdims=True) * inv_c + EPS_DIAG
    b = jnp.sum(xc_r * xc_i, axis=0, keepdims=True) * inv_c
    c = jnp.sum(xc_i * xc_i, axis=0, keepdims=True) * inv_c + EPS_DIAG

    det = a * c - b * b
    r1 = jax.lax.rsqrt(det)
    s = det * r1
    r2 = jax.lax.rsqrt(a + c + 2.0 * s)
    f = r1 * r2
    w00 = (c + s) * f
    w01 = -b * f
    w11 = (a + s) * f

    z_r = xc_r * w00 + xc_i * w01
    z_i = xc_r * w01 + xc_i * w11

    g00 = wb_ref[:, 0:1]
    g01 = wb_ref[:, 1:2]
    g10 = wb_ref[:, 2:3]
    g11 = wb_ref[:, 3:4]
    b0 = wb_ref[:, 4:5]
    b1 = wb_ref[:, 5:6]
    yr_ref[0] = z_r * g00 + z_i * g10 + b0
    yi_ref[0] = z_r * g01 + z_i * g11 + b1


@jax.jit
def kernel(x_real, x_imag, weight, bias):
    B, C, T = x_real.shape
    w4 = jnp.transpose(weight, (2, 0, 1)).reshape(C, 4)
    b2 = jnp.transpose(bias, (1, 0))
    wb = jnp.concatenate([w4, b2], axis=1)

    grid = (B, T // T_TILE)
    x_spec = pl.BlockSpec((1, C, T), lambda b, t: (b, 0, 0))
    wb_spec = pl.BlockSpec((C, 6), lambda b, t: (0, 0))
    y_spec = pl.BlockSpec((1, C, T_TILE), lambda b, t: (b, 0, t))

    yr, yi = pl.pallas_call(
        _cln_kernel,
        grid=grid,
        in_specs=[x_spec, x_spec, wb_spec],
        out_specs=[y_spec, y_spec],
        out_shape=[
            jax.ShapeDtypeStruct((B, C, T), x_real.dtype),
            jax.ShapeDtypeStruct((B, C, T), x_real.dtype),
        ],
        compiler_params=pltpu.CompilerParams(
            dimension_semantics=("parallel", "arbitrary")
        ),
    )(x_real, x_imag, wb)
    return yr, yi


# speedup vs baseline: 1.2602x; 1.2602x over previous
import jax
import jax.numpy as jnp
from jax.experimental import pallas as pl
from jax.experimental.pallas import tpu as pltpu

EPS_DIAG = 1e-06
T_TILE = 2048


def _cln_kernel(xr_ref, xi_ref, w_ref, b_ref, yr_ref, yi_ref):
    xr = xr_ref[0]
    xi = xi_ref[0]
    c_dim = xr.shape[0]
    inv_c = 1.0 / c_dim

    mu_r = jnp.sum(xr, axis=0, keepdims=True) * inv_c
    mu_i = jnp.sum(xi, axis=0, keepdims=True) * inv_c
    xc_r = xr - mu_r
    xc_i = xi - mu_i

    a = jnp.sum(xc_r * xc_r, axis=0, keepdims=True) * inv_c + EPS_DIAG
    b = jnp.sum(xc_r * xc_i, axis=0, keepdims=True) * inv_c
    c = jnp.sum(xc_i * xc_i, axis=0, keepdims=True) * inv_c + EPS_DIAG

    det = a * c - b * b
    r1 = jax.lax.rsqrt(det)
    s = det * r1
    r2 = jax.lax.rsqrt(a + c + 2.0 * s)
    f = r1 * r2
    w00 = (c + s) * f
    w01 = -b * f
    w11 = (a + s) * f

    z_r = xc_r * w00 + xc_i * w01
    z_i = xc_r * w01 + xc_i * w11

    wt = jnp.transpose(w_ref[...])
    bt = jnp.transpose(b_ref[...])
    g00 = wt[:, 0:1]
    g01 = wt[:, 1:2]
    g10 = wt[:, 2:3]
    g11 = wt[:, 3:4]
    b0 = bt[:, 0:1]
    b1 = bt[:, 1:2]
    yr_ref[0] = z_r * g00 + z_i * g10 + b0
    yi_ref[0] = z_r * g01 + z_i * g11 + b1


@jax.jit
def kernel(x_real, x_imag, weight, bias):
    B, C, T = x_real.shape
    w4 = weight.reshape(4, C)

    grid = (B, T // T_TILE)
    x_spec = pl.BlockSpec((1, C, T_TILE), lambda b, t: (b, 0, t))
    w_spec = pl.BlockSpec((4, C), lambda b, t: (0, 0))
    b_spec = pl.BlockSpec((2, C), lambda b, t: (0, 0))

    yr, yi = pl.pallas_call(
        _cln_kernel,
        grid=grid,
        in_specs=[x_spec, x_spec, w_spec, b_spec],
        out_specs=[x_spec, x_spec],
        out_shape=[
            jax.ShapeDtypeStruct((B, C, T), x_real.dtype),
            jax.ShapeDtypeStruct((B, C, T), x_real.dtype),
        ],
        compiler_params=pltpu.CompilerParams(
            dimension_semantics=("parallel", "parallel")
        ),
    )(x_real, x_imag, w4, bias)
    return yr, yi


# speedup vs baseline: 1.3255x; 1.0518x over previous
import jax
import jax.numpy as jnp
from jax.experimental import pallas as pl
from jax.experimental.pallas import tpu as pltpu

EPS_DIAG = 1e-06
T_TILE = 2048
W_CHUNK = 128


def _cln_kernel(xr_ref, xi_ref, w_ref, b_ref, yr_ref, yi_ref):
    wt = jnp.transpose(w_ref[...])
    bt = jnp.transpose(b_ref[...])
    g00 = wt[:, 0:1]
    g01 = wt[:, 1:2]
    g10 = wt[:, 2:3]
    g11 = wt[:, 3:4]
    b0 = bt[:, 0:1]
    b1 = bt[:, 1:2]

    for k in range(T_TILE // W_CHUNK):
        sl = slice(k * W_CHUNK, (k + 1) * W_CHUNK)
        xr = xr_ref[0, :, sl]
        xi = xi_ref[0, :, sl]
        inv_c = 1.0 / xr.shape[0]

        mu_r = jnp.sum(xr, axis=0, keepdims=True) * inv_c
        mu_i = jnp.sum(xi, axis=0, keepdims=True) * inv_c
        m_rr = jnp.sum(xr * xr, axis=0, keepdims=True) * inv_c
        m_ri = jnp.sum(xr * xi, axis=0, keepdims=True) * inv_c
        m_ii = jnp.sum(xi * xi, axis=0, keepdims=True) * inv_c

        a = m_rr - mu_r * mu_r + EPS_DIAG
        b = m_ri - mu_r * mu_i
        c = m_ii - mu_i * mu_i + EPS_DIAG

        det = a * c - b * b
        r1 = jax.lax.rsqrt(det)
        s = det * r1
        r2 = jax.lax.rsqrt(a + c + 2.0 * s)
        f = r1 * r2
        w00 = (c + s) * f
        w01 = -b * f
        w11 = (a + s) * f

        cr = mu_r * w00 + mu_i * w01
        ci = mu_r * w01 + mu_i * w11

        z_r = xr * w00 + xi * w01 - cr
        z_i = xr * w01 + xi * w11 - ci

        yr_ref[0, :, sl] = z_r * g00 + z_i * g10 + b0
        yi_ref[0, :, sl] = z_r * g01 + z_i * g11 + b1


@jax.jit
def kernel(x_real, x_imag, weight, bias):
    B, C, T = x_real.shape
    w4 = weight.reshape(4, C)

    grid = (B, T // T_TILE)
    x_spec = pl.BlockSpec((1, C, T_TILE), lambda b, t: (b, 0, t))
    w_spec = pl.BlockSpec((4, C), lambda b, t: (0, 0))
    b_spec = pl.BlockSpec((2, C), lambda b, t: (0, 0))

    yr, yi = pl.pallas_call(
        _cln_kernel,
        grid=grid,
        in_specs=[x_spec, x_spec, w_spec, b_spec],
        out_specs=[x_spec, x_spec],
        out_shape=[
            jax.ShapeDtypeStruct((B, C, T), x_real.dtype),
            jax.ShapeDtypeStruct((B, C, T), x_real.dtype),
        ],
        compiler_params=pltpu.CompilerParams(
            dimension_semantics=("parallel", "parallel")
        ),
    )(x_real, x_imag, w4, bias)
    return yr, yi


# speedup vs baseline: 1.3770x; 1.0389x over previous
import jax
import jax.numpy as jnp
from jax.experimental import pallas as pl
from jax.experimental.pallas import tpu as pltpu

EPS_DIAG = 1e-06
T_TILE = 2048
W_CHUNK = 128
C_GROUP = 64


def _cln_kernel(xr_ref, xi_ref, w_ref, b_ref, yr_ref, yi_ref):
    wt = jnp.transpose(w_ref[...])
    bt = jnp.transpose(b_ref[...])
    g00 = wt[:, 0:1]
    g01 = wt[:, 1:2]
    g10 = wt[:, 2:3]
    g11 = wt[:, 3:4]
    b0 = bt[:, 0:1]
    b1 = bt[:, 1:2]

    c_total = xr_ref.shape[1]
    inv_c = 1.0 / c_total
    n_groups = c_total // C_GROUP

    for k in range(T_TILE // W_CHUNK):
        sl = slice(k * W_CHUNK, (k + 1) * W_CHUNK)

        acc = [None] * 5
        for g in range(n_groups):
            xr_g = xr_ref[0, g * C_GROUP:(g + 1) * C_GROUP, sl]
            xi_g = xi_ref[0, g * C_GROUP:(g + 1) * C_GROUP, sl]
            shape3 = (C_GROUP // 8, 8, W_CHUNK)
            for idx, prod in enumerate((
                xr_g, xi_g, xr_g * xr_g, xr_g * xi_g, xi_g * xi_g,
            )):
                part = jnp.sum(prod.reshape(shape3), axis=0)
                acc[idx] = part if acc[idx] is None else acc[idx] + part

        def _col(v):
            return jnp.sum(v, axis=0, keepdims=True)

        mu_r = _col(acc[0]) * inv_c
        mu_i = _col(acc[1]) * inv_c
        m_rr = _col(acc[2]) * inv_c
        m_ri = _col(acc[3]) * inv_c
        m_ii = _col(acc[4]) * inv_c

        a = m_rr - mu_r * mu_r + EPS_DIAG
        b = m_ri - mu_r * mu_i
        c = m_ii - mu_i * mu_i + EPS_DIAG

        det = a * c - b * b
        r1 = jax.lax.rsqrt(det)
        s = det * r1
        r2 = jax.lax.rsqrt(a + c + 2.0 * s)
        f = r1 * r2
        w00 = (c + s) * f
        w01 = -b * f
        w11 = (a + s) * f

        cr = mu_r * w00 + mu_i * w01
        ci = mu_r * w01 + mu_i * w11

        for g in range(n_groups):
            cs = slice(g * C_GROUP, (g + 1) * C_GROUP)
            xr_g = xr_ref[0, cs, sl]
            xi_g = xi_ref[0, cs, sl]
            z_r = xr_g * w00 + xi_g * w01 - cr
            z_i = xr_g * w01 + xi_g * w11 - ci
            yr_ref[0, cs, sl] = z_r * g00[cs] + z_i * g10[cs] + b0[cs]
            yi_ref[0, cs, sl] = z_r * g01[cs] + z_i * g11[cs] + b1[cs]


@jax.jit
def kernel(x_real, x_imag, weight, bias):
    B, C, T = x_real.shape
    w4 = weight.reshape(4, C)

    grid = (B, T // T_TILE)
    x_spec = pl.BlockSpec((1, C, T_TILE), lambda b, t: (b, 0, t))
    w_spec = pl.BlockSpec((4, C), lambda b, t: (0, 0))
    b_spec = pl.BlockSpec((2, C), lambda b, t: (0, 0))

    yr, yi = pl.pallas_call(
        _cln_kernel,
        grid=grid,
        in_specs=[x_spec, x_spec, w_spec, b_spec],
        out_specs=[x_spec, x_spec],
        out_shape=[
            jax.ShapeDtypeStruct((B, C, T), x_real.dtype),
            jax.ShapeDtypeStruct((B, C, T), x_real.dtype),
        ],
        compiler_params=pltpu.CompilerParams(
            dimension_semantics=("parallel", "parallel")
        ),
    )(x_real, x_imag, w4, bias)
    return yr, yi


# speedup vs baseline: 1.4464x; 1.0504x over previous
import jax
import jax.numpy as jnp
from jax.experimental import pallas as pl
from jax.experimental.pallas import tpu as pltpu

EPS_DIAG = 1e-06
T_TILE = 4096
W_CHUNK = 128
C_GROUP = 64


def _cln_kernel(xr_ref, xi_ref, w_ref, b_ref, yr_ref, yi_ref):
    wt = jnp.transpose(w_ref[...])
    bt = jnp.transpose(b_ref[...])
    g00 = wt[:, 0:1]
    g01 = wt[:, 1:2]
    g10 = wt[:, 2:3]
    g11 = wt[:, 3:4]
    b0 = bt[:, 0:1]
    b1 = bt[:, 1:2]

    c_total = xr_ref.shape[1]
    inv_c = 1.0 / c_total
    n_groups = c_total // C_GROUP
    n_chunks = T_TILE // W_CHUNK
    shape3 = (C_GROUP // 8, 8, W_CHUNK)

    for k in range(n_chunks):
        sl = slice(k * W_CHUNK, (k + 1) * W_CHUNK)
        acc = [None] * 5
        for g in range(n_groups):
            cs = slice(g * C_GROUP, (g + 1) * C_GROUP)
            xr_g = xr_ref[0, cs, sl]
            xi_g = xi_ref[0, cs, sl]
            for idx, prod in enumerate((
                xr_g, xi_g, xr_g * xr_g, xr_g * xi_g, xi_g * xi_g,
            )):
                part = jnp.sum(prod.reshape(shape3), axis=0)
                acc[idx] = part if acc[idx] is None else acc[idx] + part

        def _col(v):
            return jnp.sum(v, axis=0, keepdims=True)

        mu_r = _col(acc[0]) * inv_c
        mu_i = _col(acc[1]) * inv_c
        m_rr = _col(acc[2]) * inv_c
        m_ri = _col(acc[3]) * inv_c
        m_ii = _col(acc[4]) * inv_c

        a = m_rr - mu_r * mu_r + EPS_DIAG
        b = m_ri - mu_r * mu_i
        c = m_ii - mu_i * mu_i + EPS_DIAG

        det = a * c - b * b
        r1 = jax.lax.rsqrt(det)
        s = det * r1
        r2 = jax.lax.rsqrt(a + c + 2.0 * s)
        f = r1 * r2
        w00 = (c + s) * f
        w01 = -b * f
        w11 = (a + s) * f

        cr = mu_r * w00 + mu_i * w01
        ci = mu_r * w01 + mu_i * w11

        for g in range(n_groups):
            cs = slice(g * C_GROUP, (g + 1) * C_GROUP)
            xr_g = xr_ref[0, cs, sl]
            xi_g = xi_ref[0, cs, sl]
            z_r = xr_g * w00 + xi_g * w01 - cr
            z_i = xr_g * w01 + xi_g * w11 - ci
            yr_ref[0, cs, sl] = z_r * g00[cs] + z_i * g10[cs] + b0[cs]
            yi_ref[0, cs, sl] = z_r * g01[cs] + z_i * g11[cs] + b1[cs]


@jax.jit
def kernel(x_real, x_imag, weight, bias):
    B, C, T = x_real.shape
    w4 = weight.reshape(4, C)

    grid = (B, T // T_TILE)
    x_spec = pl.BlockSpec((1, C, T_TILE), lambda b, t: (b, 0, t))
    w_spec = pl.BlockSpec((4, C), lambda b, t: (0, 0))
    b_spec = pl.BlockSpec((2, C), lambda b, t: (0, 0))

    yr, yi = pl.pallas_call(
        _cln_kernel,
        grid=grid,
        in_specs=[x_spec, x_spec, w_spec, b_spec],
        out_specs=[x_spec, x_spec],
        out_shape=[
            jax.ShapeDtypeStruct((B, C, T), x_real.dtype),
            jax.ShapeDtypeStruct((B, C, T), x_real.dtype),
        ],
        compiler_params=pltpu.CompilerParams(
            dimension_semantics=("parallel", "parallel")
        ),
    )(x_real, x_imag, w4, bias)
    return yr, yi


# speedup vs baseline: 1.4827x; 1.0251x over previous
import jax
import jax.numpy as jnp
from jax.experimental import pallas as pl
from jax.experimental.pallas import tpu as pltpu

EPS_DIAG = 1e-06
T_TILE = 4096
W_CHUNK = 128
C_GROUP = 64
W_CHUNK2 = 512
C_GROUP2 = 32


def _cln_kernel(xr_ref, xi_ref, w_ref, b_ref, yr_ref, yi_ref):
    wt = jnp.transpose(w_ref[...])
    bt = jnp.transpose(b_ref[...])
    g00 = wt[:, 0:1]
    g01 = wt[:, 1:2]
    g10 = wt[:, 2:3]
    g11 = wt[:, 3:4]
    b0 = bt[:, 0:1]
    b1 = bt[:, 1:2]

    c_total = xr_ref.shape[1]
    inv_c = 1.0 / c_total
    n_groups = c_total // C_GROUP
    n_chunks = T_TILE // W_CHUNK
    shape3 = (C_GROUP // 8, 8, W_CHUNK)

    rows = []
    for k in range(n_chunks):
        sl = slice(k * W_CHUNK, (k + 1) * W_CHUNK)
        acc = [None] * 5
        for g in range(n_groups):
            cs = slice(g * C_GROUP, (g + 1) * C_GROUP)
            xr_g = xr_ref[0, cs, sl]
            xi_g = xi_ref[0, cs, sl]
            for idx, prod in enumerate((
                xr_g, xi_g, xr_g * xr_g, xr_g * xi_g, xi_g * xi_g,
            )):
                part = jnp.sum(prod.reshape(shape3), axis=0)
                acc[idx] = part if acc[idx] is None else acc[idx] + part

        def _col(v):
            return jnp.sum(v, axis=0, keepdims=True)

        mu_r = _col(acc[0]) * inv_c
        mu_i = _col(acc[1]) * inv_c
        m_rr = _col(acc[2]) * inv_c
        m_ri = _col(acc[3]) * inv_c
        m_ii = _col(acc[4]) * inv_c

        a = m_rr - mu_r * mu_r + EPS_DIAG
        b = m_ri - mu_r * mu_i
        c = m_ii - mu_i * mu_i + EPS_DIAG

        det = a * c - b * b
        r1 = jax.lax.rsqrt(det)
        s = det * r1
        r2 = jax.lax.rsqrt(a + c + 2.0 * s)
        f = r1 * r2
        w00 = (c + s) * f
        w01 = -b * f
        w11 = (a + s) * f

        cr = mu_r * w00 + mu_i * w01
        ci = mu_r * w01 + mu_i * w11
        rows.append((w00, w01, w11, cr, ci))

    n_chunks2 = T_TILE // W_CHUNK2
    ratio = W_CHUNK2 // W_CHUNK
    for g in range(c_total // C_GROUP2):
        cs = slice(g * C_GROUP2, (g + 1) * C_GROUP2)
        gg00, gg01, gg10, gg11 = g00[cs], g01[cs], g10[cs], g11[cs]
        bb0, bb1 = b0[cs], b1[cs]
        for k2 in range(n_chunks2):
            sl = slice(k2 * W_CHUNK2, (k2 + 1) * W_CHUNK2)
            sub = rows[k2 * ratio:(k2 + 1) * ratio]
            w00 = jnp.concatenate([r[0] for r in sub], axis=1)
            w01 = jnp.concatenate([r[1] for r in sub], axis=1)
            w11 = jnp.concatenate([r[2] for r in sub], axis=1)
            cr = jnp.concatenate([r[3] for r in sub], axis=1)
            ci = jnp.concatenate([r[4] for r in sub], axis=1)
            xr_g = xr_ref[0, cs, sl]
            xi_g = xi_ref[0, cs, sl]
            z_r = xr_g * w00 + xi_g * w01 - cr
            z_i = xr_g * w01 + xi_g * w11 - ci
            yr_ref[0, cs, sl] = z_r * gg00 + z_i * gg10 + bb0
            yi_ref[0, cs, sl] = z_r * gg01 + z_i * gg11 + bb1


@jax.jit
def kernel(x_real, x_imag, weight, bias):
    B, C, T = x_real.shape
    w4 = weight.reshape(4, C)

    grid = (B, T // T_TILE)
    x_spec = pl.BlockSpec((1, C, T_TILE), lambda b, t: (b, 0, t))
    w_spec = pl.BlockSpec((4, C), lambda b, t: (0, 0))
    b_spec = pl.BlockSpec((2, C), lambda b, t: (0, 0))

    yr, yi = pl.pallas_call(
        _cln_kernel,
        grid=grid,
        in_specs=[x_spec, x_spec, w_spec, b_spec],
        out_specs=[x_spec, x_spec],
        out_shape=[
            jax.ShapeDtypeStruct((B, C, T), x_real.dtype),
            jax.ShapeDtypeStruct((B, C, T), x_real.dtype),
        ],
        compiler_params=pltpu.CompilerParams(
            dimension_semantics=("parallel", "parallel")
        ),
    )(x_real, x_imag, w4, bias)
    return yr, yi
